# baseline (speedup 1.0000x reference)
import functools

import jax
import jax.numpy as jnp
from jax import lax
from jax.experimental import pallas as pl
from jax.experimental.pallas import tpu as pltpu
from jax.experimental.pallas import tpu_sc as plsc

_B = 4096
_I = 200
_H = 64


def _make_sc_gather(B, D):
    info = plsc.get_sparse_core_info()
    NC, NS = info.num_cores, info.num_subcores
    NW = NC * NS
    b_per_w = B // NW
    mesh = plsc.VectorSubcoreMesh(core_axis_name="c", subcore_axis_name="s")

    @functools.partial(
        pl.kernel,
        mesh=mesh,
        out_type=jax.ShapeDtypeStruct((B, D), jnp.float32),
        scratch_types=[
            pltpu.VMEM((b_per_w,), jnp.int32),
            pltpu.VMEM((b_per_w, D), jnp.float32),
            pltpu.SemaphoreType.DMA,
        ],
    )
    def gather_kernel(idx_hbm, table_hbm, out_hbm, idx_v, rows_v, sem):
        wid = lax.axis_index("s") * NC + lax.axis_index("c")
        base = wid * b_per_w
        pltpu.sync_copy(idx_hbm.at[pl.ds(base, b_per_w)], idx_v)
        pltpu.async_copy(table_hbm.at[idx_v], rows_v, sem).wait()
        pltpu.sync_copy(rows_v, out_hbm.at[pl.ds(base, b_per_w)])

    return gather_kernel


def _bmm_body(iemb_ref, upair_ref, par_ref, out_ref):
    upair = upair_ref[...]
    lo = upair[:, :_H]
    hi = upair[:, _H:]
    par = par_ref[...]
    uemb = lo + par * (hi - lo)
    out_ref[...] = jnp.sum(iemb_ref[...] * uemb[:, None, :], axis=-1)


def _tc_bmm(iemb, upair, par, block_b=256):
    B, I, H = iemb.shape
    grid = (B // block_b,)
    return pl.pallas_call(
        _bmm_body,
        grid=grid,
        in_specs=[
            pl.BlockSpec((block_b, I, H), lambda i: (i, 0, 0)),
            pl.BlockSpec((block_b, 2 * H), lambda i: (i, 0)),
            pl.BlockSpec((block_b, 1), lambda i: (i, 0)),
        ],
        out_specs=pl.BlockSpec((block_b, I), lambda i: (i, 0)),
        out_shape=jax.ShapeDtypeStruct((B, I), jnp.float32),
    )(iemb, upair, par)


def kernel(userid_input, iemb, uembedding_weight):
    idx = userid_input.reshape(-1)
    table2 = uembedding_weight.reshape(-1, 2 * _H)
    par = (userid_input & 1).astype(jnp.float32)
    gather = _make_sc_gather(_B, 2 * _H)
    upair = gather(idx >> 1, table2)
    return _tc_bmm(iemb, upair, par)


# speedup vs baseline: 3.9547x; 3.9547x over previous
import functools

import jax
import jax.numpy as jnp
from jax import lax
from jax.experimental import pallas as pl
from jax.experimental.pallas import tpu as pltpu
from jax.experimental.pallas import tpu_sc as plsc

_B = 4096
_I = 200
_H = 64
_LANES = 16
_NSLOT = 8


def _make_sc_gather():
    info = plsc.get_sparse_core_info()
    NC, NS = info.num_cores, info.num_subcores
    NW = NC * NS
    bpw = _B // NW
    mesh = plsc.VectorSubcoreMesh(core_axis_name="c", subcore_axis_name="s")

    @functools.partial(
        pl.kernel,
        mesh=mesh,
        out_type=jax.ShapeDtypeStruct((_H, _B), jnp.float32),
        scratch_types=[
            pltpu.VMEM((bpw,), jnp.int32),
            pltpu.VMEM((_NSLOT, _H, 128), jnp.float32),
            pltpu.VMEM_SHARED((NS, _H, 128), jnp.float32),
            pltpu.SemaphoreType.DMA((_NSLOT,)),
            pltpu.SemaphoreType.DMA((_NSLOT,)),
        ],
    )
    def gather_kernel(
        idx_hbm, tableT_hbm, out_hbm, idx_v, ring_v, sh_v, tsems, csems
    ):
        cid = lax.axis_index("c")
        sid = lax.axis_index("s")
        wid = sid * NC + cid
        base = wid * bpw
        my_sh = sh_v.at[sid]

        pltpu.sync_copy(idx_hbm.at[pl.ds(base, bpw)], idx_v)

        def fire_tile(u, k):
            start = pl.multiple_of((u >> 7) * 128, 128)
            pltpu.async_copy(
                tableT_hbm.at[:, pl.ds(start, 128)], ring_v.at[k], tsems.at[k]
            )

        def wait_tile(k):
            pltpu.make_async_copy(
                tableT_hbm.at[:, pl.ds(0, 128)], ring_v.at[k], tsems.at[k]
            ).wait()

        def fire_col(u, b, k):
            pltpu.async_copy(
                ring_v.at[k].at[:, pl.ds(u & 127, 1)],
                my_sh.at[:, pl.ds(b, 1)],
                csems.at[k],
            )

        def wait_col(k):
            pltpu.make_async_copy(
                ring_v.at[k].at[:, pl.ds(0, 1)],
                my_sh.at[:, pl.ds(0, 1)],
                csems.at[k],
            ).wait()

        @pl.loop(0, bpw // _LANES)
        def group(g):
            v = idx_v[pl.ds(g * _LANES, _LANES)]
            for j in range(_LANES):
                k = j % _NSLOT
                fire_tile(v[j], k)
                wait_tile(k)
                fire_col(v[j], g * _LANES + j, k)
                wait_col(k)

        pltpu.sync_copy(my_sh, out_hbm.at[:, pl.ds(base, bpw)])

    return gather_kernel


def _bmm_body(iembT_ref, uemb_ref, out_ref):
    out_ref[...] = jnp.sum(iembT_ref[...] * uemb_ref[...][None, :, :], axis=1)


def _tc_bmm(iembT, uembT, block_i=8):
    I, H, B = iembT.shape
    return pl.pallas_call(
        _bmm_body,
        grid=(I // block_i,),
        in_specs=[
            pl.BlockSpec((block_i, H, B), lambda i: (i, 0, 0)),
            pl.BlockSpec((H, B), lambda i: (0, 0)),
        ],
        out_specs=pl.BlockSpec((block_i, B), lambda i: (i, 0)),
        out_shape=jax.ShapeDtypeStruct((I, B), jnp.float32),
    )(iembT, uembT)


def kernel(userid_input, iemb, uembedding_weight):
    idx = userid_input.reshape(-1)
    tableT = uembedding_weight.T
    iembT = jnp.transpose(iemb, (1, 2, 0))
    gather = _make_sc_gather()
    uembT = gather(idx, tableT)
    predT = _tc_bmm(iembT, uembT)
    return predT.T


# speedup vs baseline: 7.2207x; 1.8259x over previous
import functools

import jax
import jax.numpy as jnp
from jax import lax
from jax.experimental import pallas as pl
from jax.experimental.pallas import tpu as pltpu
from jax.experimental.pallas import tpu_sc as plsc

_B = 4096
_I = 200
_H = 64
_LANES = 16
_NSLOT = 8


def _make_sc_gather():
    info = plsc.get_sparse_core_info()
    NC, NS = info.num_cores, info.num_subcores
    NW = NC * NS
    bpw = _B // NW
    mesh = plsc.VectorSubcoreMesh(core_axis_name="c", subcore_axis_name="s")

    @functools.partial(
        pl.kernel,
        mesh=mesh,
        out_type=jax.ShapeDtypeStruct((_H, _B), jnp.float32),
        scratch_types=[
            pltpu.VMEM((bpw,), jnp.int32),
            pltpu.VMEM((_NSLOT, _H, 128), jnp.float32),
            pltpu.VMEM_SHARED((NS, _H, 128), jnp.float32),
            pltpu.SemaphoreType.DMA((_NSLOT,)),
            pltpu.SemaphoreType.DMA((_NSLOT,)),
        ],
    )
    def gather_kernel(
        idx_hbm, tableT_hbm, out_hbm, idx_v, ring_v, sh_v, tsems, csems
    ):
        cid = lax.axis_index("c")
        sid = lax.axis_index("s")
        wid = sid * NC + cid
        base = wid * bpw
        my_sh = sh_v.at[sid]

        pltpu.sync_copy(idx_hbm.at[pl.ds(base, bpw)], idx_v)

        def fire_tile(u, k):
            start = pl.multiple_of((u >> 7) * 128, 128)
            pltpu.async_copy(
                tableT_hbm.at[:, pl.ds(start, 128)], ring_v.at[k], tsems.at[k]
            )

        def wait_tile(k):
            pltpu.make_async_copy(
                tableT_hbm.at[:, pl.ds(0, 128)], ring_v.at[k], tsems.at[k]
            ).wait()

        def fire_col(u, b, k):
            pltpu.async_copy(
                ring_v.at[k].at[:, pl.ds(u & 127, 1)],
                my_sh.at[:, pl.ds(b, 1)],
                csems.at[k],
            )

        def wait_col(k):
            pltpu.make_async_copy(
                ring_v.at[k].at[:, pl.ds(0, 1)],
                my_sh.at[:, pl.ds(0, 1)],
                csems.at[k],
            ).wait()

        vs = [idx_v[pl.ds(g * _LANES, _LANES)] for g in range(bpw // _LANES)]

        def u(b):
            return vs[b // _LANES][b % _LANES]

        _LAG = 6
        for b in range(bpw + _LAG):
            if b < bpw:
                if b >= _NSLOT:
                    wait_col(b % _NSLOT)
                fire_tile(u(b), b % _NSLOT)
            if b >= _LAG:
                bb = b - _LAG
                wait_tile(bb % _NSLOT)
                fire_col(u(bb), bb, bb % _NSLOT)
        for k in range(_NSLOT):
            wait_col(k)

        pltpu.sync_copy(my_sh, out_hbm.at[:, pl.ds(base, bpw)])

    return gather_kernel


def _bmm_body(iembT_ref, uemb_ref, out_ref):
    out_ref[...] = jnp.sum(iembT_ref[...] * uemb_ref[...][None, :, :], axis=1)


def _tc_bmm(iembT, uembT, block_i=8):
    I, H, B = iembT.shape
    return pl.pallas_call(
        _bmm_body,
        grid=(I // block_i,),
        in_specs=[
            pl.BlockSpec((block_i, H, B), lambda i: (i, 0, 0)),
            pl.BlockSpec((H, B), lambda i: (0, 0)),
        ],
        out_specs=pl.BlockSpec((block_i, B), lambda i: (i, 0)),
        out_shape=jax.ShapeDtypeStruct((I, B), jnp.float32),
    )(iembT, uembT)


def kernel(userid_input, iemb, uembedding_weight):
    idx = userid_input.reshape(-1)
    tableT = uembedding_weight.T
    iembT = jnp.transpose(iemb, (1, 2, 0))
    gather = _make_sc_gather()
    uembT = gather(idx, tableT)
    predT = _tc_bmm(iembT, uembT)
    return predT.T
